# baseline (speedup 1.0000x reference)
import functools

import jax
import jax.numpy as jnp
from jax import lax
from jax.experimental import pallas as pl
from jax.experimental.pallas import tpu as pltpu
from jax.experimental.pallas import tpu_sc as plsc

_BATCH = 256
_RES = 65536
_DIM_OUT = 1024
_M = _RES // _DIM_OUT
_HALF = _DIM_OUT // 2


def _readout_body(x_hbm, idx_hbm, out_hbm, idx_v, row_v, out_v, idx_sem):
    info = plsc.get_sparse_core_info()
    nc = info.num_cores
    nw = nc * info.num_subcores
    rows_per_w = _BATCH // nw
    wid = lax.axis_index("s") * nc + lax.axis_index("c")
    row0 = wid * rows_per_w

    idx_cp = pltpu.async_copy(idx_hbm, idx_v, idx_sem)
    pltpu.sync_copy(x_hbm.at[row0], row_v)
    idx_cp.wait()

    def gather_row(r):
        def do_group(kb, _):
            base = kb * 16

            def gather_m(mm, accs):
                a0, a1, b0, b1 = accs
                w0 = idx_v[pl.ds((2 * mm) * _HALF + base, 16)]
                w1 = idx_v[pl.ds((2 * mm + 1) * _HALF + base, 16)]
                a0 = a0 + plsc.load_gather(row_v, [w0 & 0xFFFF])
                a1 = a1 + plsc.load_gather(
                    row_v, [lax.shift_right_logical(w0, 16)])
                b0 = b0 + plsc.load_gather(row_v, [w1 & 0xFFFF])
                b1 = b1 + plsc.load_gather(
                    row_v, [lax.shift_right_logical(w1, 16)])
                return a0, a1, b0, b1

            zero = jnp.zeros((16,), jnp.float32)
            a0, a1, b0, b1 = lax.fori_loop(0, _M // 2, gather_m,
                                           (zero, zero, zero, zero),
                                           unroll=8)
            acc0 = a0 + b0
            acc1 = a1 + b1
            off = r * _DIM_OUT + base
            out_v[pl.ds(off, 16)] = jnp.where(acc0 > 0.5, 1.0, acc0)
            out_v[pl.ds(off + _HALF, 16)] = jnp.where(acc1 > 0.5, 1.0, acc1)
            return 0

        lax.fori_loop(0, _HALF // 16, do_group, 0)

    gather_row(0)

    def do_row(r, _):
        pltpu.sync_copy(x_hbm.at[row0 + r], row_v)
        gather_row(r)
        return 0

    lax.fori_loop(1, rows_per_w, do_row, 0)
    pltpu.sync_copy(out_v, out_hbm.at[pl.ds(row0 * _DIM_OUT,
                                            rows_per_w * _DIM_OUT)])


@jax.jit
def _readout(x, idx_packed):
    mesh = plsc.VectorSubcoreMesh(core_axis_name="c", subcore_axis_name="s")
    k = functools.partial(
        pl.kernel,
        mesh=mesh,
        out_type=jax.ShapeDtypeStruct((_BATCH * _DIM_OUT,), jnp.float32),
        scratch_types=[
            pltpu.VMEM((_RES // 2,), jnp.int32),
            pltpu.VMEM((_RES,), jnp.float32),
            pltpu.VMEM((8 * _DIM_OUT,), jnp.float32),
            pltpu.SemaphoreType.DMA,
        ],
        compiler_params=pltpu.CompilerParams(needs_layout_passes=False),
    )(_readout_body)
    return k(x, idx_packed)


def kernel(x, pre, post):
    del post
    p = pre.reshape(_M, _DIM_OUT)
    packed = p[:, :_HALF] | (p[:, _HALF:] << 16)
    return _readout(x, packed.reshape(-1)).reshape(_BATCH, _DIM_OUT)


# speedup vs baseline: 1.0030x; 1.0030x over previous
import functools

import jax
import jax.numpy as jnp
from jax import lax
from jax.experimental import pallas as pl
from jax.experimental.pallas import tpu as pltpu
from jax.experimental.pallas import tpu_sc as plsc

_BATCH = 256
_RES = 65536
_DIM_OUT = 1024
_M = _RES // _DIM_OUT
_HALF = _DIM_OUT // 2


def _readout_body(x_hbm, idx_hbm, out_hbm, idx_v, row_v, out_v, idx_sem):
    info = plsc.get_sparse_core_info()
    nc = info.num_cores
    nw = nc * info.num_subcores
    rows_per_w = _BATCH // nw
    wid = lax.axis_index("s") * nc + lax.axis_index("c")
    row0 = wid * rows_per_w

    idx_cp = pltpu.async_copy(idx_hbm, idx_v, idx_sem)
    pltpu.sync_copy(x_hbm.at[row0], row_v)
    idx_cp.wait()

    def gather_row(r):
        def do_group(kb, _):
            base = kb * 16

            def gather_m(mm, accs):
                a0, a1, b0, b1 = accs
                w0 = idx_v[pl.ds((2 * mm) * _HALF + base, 16)]
                w1 = idx_v[pl.ds((2 * mm + 1) * _HALF + base, 16)]
                a0 = a0 + plsc.load_gather(row_v, [w0 & 0xFFFF])
                a1 = a1 + plsc.load_gather(
                    row_v, [lax.shift_right_logical(w0, 16)])
                b0 = b0 + plsc.load_gather(row_v, [w1 & 0xFFFF])
                b1 = b1 + plsc.load_gather(
                    row_v, [lax.shift_right_logical(w1, 16)])
                return a0, a1, b0, b1

            zero = jnp.zeros((16,), jnp.float32)
            a0, a1, b0, b1 = lax.fori_loop(0, _M // 2, gather_m,
                                           (zero, zero, zero, zero),
                                           unroll=8)
            acc0 = a0 + b0
            acc1 = a1 + b1
            off = r * _DIM_OUT + base
            out_v[pl.ds(off, 16)] = jnp.where(acc0 > 0.5, 1.0, acc0)
            out_v[pl.ds(off + _HALF, 16)] = jnp.where(acc1 > 0.5, 1.0, acc1)
            return 0

        lax.fori_loop(0, _HALF // 16, do_group, 0)

    def do_row(r, _):
        @pl.when(r > 0)
        def _():
            pltpu.sync_copy(x_hbm.at[row0 + r], row_v)

        gather_row(r)
        return 0

    lax.fori_loop(0, rows_per_w, do_row, 0)
    pltpu.sync_copy(out_v, out_hbm.at[pl.ds(row0 * _DIM_OUT,
                                            rows_per_w * _DIM_OUT)])


@jax.jit
def _readout(x, idx_packed):
    mesh = plsc.VectorSubcoreMesh(core_axis_name="c", subcore_axis_name="s")
    k = functools.partial(
        pl.kernel,
        mesh=mesh,
        out_type=jax.ShapeDtypeStruct((_BATCH * _DIM_OUT,), jnp.float32),
        scratch_types=[
            pltpu.VMEM((_RES // 2,), jnp.int32),
            pltpu.VMEM((_RES,), jnp.float32),
            pltpu.VMEM((8 * _DIM_OUT,), jnp.float32),
            pltpu.SemaphoreType.DMA,
        ],
        compiler_params=pltpu.CompilerParams(needs_layout_passes=False),
    )(_readout_body)
    return k(x, idx_packed)


def kernel(x, pre, post):
    del post
    p = pre.reshape(_M, _DIM_OUT)
    packed = p[:, :_HALF] | (p[:, _HALF:] << 16)
    return _readout(x, packed.reshape(-1)).reshape(_BATCH, _DIM_OUT)


# speedup vs baseline: 1.0142x; 1.0111x over previous
import functools

import jax
import jax.numpy as jnp
from jax import lax
from jax.experimental import pallas as pl
from jax.experimental.pallas import tpu as pltpu
from jax.experimental.pallas import tpu_sc as plsc

_BATCH = 256
_RES = 65536
_DIM_OUT = 1024
_M = _RES // _DIM_OUT
_HALF = _DIM_OUT // 2


def _readout_body(x_hbm, idx_hbm, out_hbm, idx_v, row_v, out_v):
    info = plsc.get_sparse_core_info()
    nc = info.num_cores
    nw = nc * info.num_subcores
    rows_per_w = _BATCH // nw
    wid = lax.axis_index("s") * nc + lax.axis_index("c")

    pltpu.sync_copy(idx_hbm, idx_v)

    def do_row(r, _):
        row = wid * rows_per_w + r
        pltpu.sync_copy(x_hbm.at[row], row_v)

        def do_group(kb, _):
            base = kb * 16

            def gather_m(mm, accs):
                a0, a1, b0, b1 = accs
                w0 = idx_v[pl.ds((2 * mm) * _HALF + base, 16)]
                w1 = idx_v[pl.ds((2 * mm + 1) * _HALF + base, 16)]
                a0 = a0 + plsc.load_gather(row_v, [w0 & 0xFFFF])
                a1 = a1 + plsc.load_gather(
                    row_v, [lax.shift_right_logical(w0, 16)])
                b0 = b0 + plsc.load_gather(row_v, [w1 & 0xFFFF])
                b1 = b1 + plsc.load_gather(
                    row_v, [lax.shift_right_logical(w1, 16)])
                return a0, a1, b0, b1

            zero = jnp.zeros((16,), jnp.float32)
            a0, a1, b0, b1 = lax.fori_loop(0, _M // 2, gather_m,
                                           (zero, zero, zero, zero),
                                           unroll=16)
            acc0 = a0 + b0
            acc1 = a1 + b1
            out_v[pl.ds(base, 16)] = jnp.where(acc0 > 0.5, 1.0, acc0)
            out_v[pl.ds(_HALF + base, 16)] = jnp.where(acc1 > 0.5, 1.0, acc1)
            return 0

        lax.fori_loop(0, _HALF // 16, do_group, 0)
        pltpu.sync_copy(out_v, out_hbm.at[row])
        return 0

    lax.fori_loop(0, rows_per_w, do_row, 0)


@jax.jit
def _readout(x, idx_packed):
    mesh = plsc.VectorSubcoreMesh(core_axis_name="c", subcore_axis_name="s")
    k = functools.partial(
        pl.kernel,
        mesh=mesh,
        out_type=jax.ShapeDtypeStruct((_BATCH, _DIM_OUT), jnp.float32),
        scratch_types=[
            pltpu.VMEM((_RES // 2,), jnp.int32),
            pltpu.VMEM((_RES,), jnp.float32),
            pltpu.VMEM((_DIM_OUT,), jnp.float32),
        ],
        compiler_params=pltpu.CompilerParams(needs_layout_passes=False),
    )(_readout_body)
    return k(x, idx_packed)


def kernel(x, pre, post):
    del post
    p = pre.reshape(_M, _DIM_OUT)
    packed = p[:, :_HALF] | (p[:, _HALF:] << 16)
    return _readout(x, packed.reshape(-1))
